# baseline (speedup 1.0000x reference)
import functools

import jax
import jax.numpy as jnp
from jax import lax
from jax.experimental import pallas as pl
from jax.experimental.pallas import tpu as pltpu
from jax.experimental.pallas import tpu_sc as plsc

_B = 4096
_L = 20
_D = 16
_NC = 2
_NS = 16
_NW = _NC * _NS
_BPW = _B // _NW
_IPW = _BPW * _L
_CH = 128
_NCH = _IPW // _CH

_mesh = plsc.VectorSubcoreMesh(core_axis_name="c", subcore_axis_name="s")


@functools.partial(
    pl.kernel,
    mesh=_mesh,
    out_type=jax.ShapeDtypeStruct((_B,), jnp.float32),
    scratch_types=[
        pltpu.VMEM((_NCH, _CH), jnp.int32),
        pltpu.VMEM((_IPW, _D), jnp.float32),
        pltpu.VMEM((32,), jnp.float32),
        pltpu.VMEM((_BPW,), jnp.float32),
        pltpu.SemaphoreType.DMA,
    ],
    compiler_params=pltpu.CompilerParams(
        needs_layout_passes=False, use_tc_tiling_on_sc=False),
)
def _sc_forward(idx_hbm, wb_hbm, table_hbm, out_hbm,
                idx_v, rows_v, wb_v, out_v, sem):
    wid = lax.axis_index("s") * _NC + lax.axis_index("c")
    base = wid * _BPW

    pltpu.sync_copy(idx_hbm.at[wid], idx_v)
    pltpu.sync_copy(wb_hbm, wb_v)

    copies = [
        pltpu.async_copy(
            table_hbm.at[idx_v.at[ch]],
            rows_v.at[pl.ds(ch * _CH, _CH)],
            sem,
        )
        for ch in range(_NCH)
    ]
    for cp in copies:
        cp.wait()

    w = wb_v[pl.ds(0, _D)]
    bias = wb_v[pl.ds(_D, 16)][0]
    lane = lax.iota(jnp.int32, 16)

    def group_body(g, _):
        yv = jnp.zeros((16,), jnp.float32)
        for k in range(16):
            p = (g * 16 + k) * _L
            acc = rows_v[p]
            for l in range(1, _L):
                acc = acc + rows_v[p + l]
            y = jnp.sum(acc * w) + bias
            yv = jnp.where(lane == k, y, yv)
        out_v[pl.ds(g * 16, 16)] = 1.0 / (1.0 + jnp.exp(-yv))
        return 0

    lax.fori_loop(0, _BPW // 16, group_body, 0)

    pltpu.sync_copy(out_v, out_hbm.at[pl.ds(base, _BPW)])


def kernel(x, table, W, b):
    idx = x.astype(jnp.int32).reshape(_NW, _NCH, _CH)
    wb = jnp.concatenate(
        [W.reshape(_D).astype(jnp.float32),
         b.astype(jnp.float32),
         jnp.zeros((32 - _D - 1,), jnp.float32)]
    )
    out = _sc_forward(idx, wb, table)
    return out.reshape(_B, 1)


# speedup vs baseline: 2.7817x; 2.7817x over previous
import functools

import jax
import jax.numpy as jnp
from jax import lax
from jax.experimental import pallas as pl
from jax.experimental.pallas import tpu as pltpu
from jax.experimental.pallas import tpu_sc as plsc

_B = 4096
_L = 20
_D = 16
_NC = 2
_NS = 16
_NW = _NC * _NS
_BPW = _B // _NW

_V = 1000000
_CPB = 4096
_GRID = (_V + _CPB - 1) // _CPB
_ROWS = _GRID * (_CPB // 128)

_mesh = plsc.VectorSubcoreMesh(core_axis_name="c", subcore_axis_name="s")


def _tdot_body(tt_ref, w_ref, b_ref, out_ref):
    acc = tt_ref[...] * w_ref[...]
    s = jnp.sum(acc, axis=0, keepdims=True)
    bias = b_ref[0] * (1.0 / _L)
    for r in range(_CPB // 128):
        out_ref[r:r + 1, :] = s[:, r * 128:(r + 1) * 128] + bias


def _tdot(table_t, W, b):
    return pl.pallas_call(
        _tdot_body,
        grid=(_GRID,),
        in_specs=[
            pl.BlockSpec((_D, _CPB), lambda g: (0, g)),
            pl.BlockSpec((_D, 1), lambda g: (0, 0)),
            pl.BlockSpec(memory_space=pltpu.SMEM),
        ],
        out_specs=pl.BlockSpec((_CPB // 128, 128), lambda g: (g, 0)),
        out_shape=jax.ShapeDtypeStruct((_ROWS, 128), jnp.float32),
    )(table_t, W, b)


@functools.partial(
    pl.kernel,
    mesh=_mesh,
    out_type=jax.ShapeDtypeStruct((_B,), jnp.float32),
    scratch_types=[
        pltpu.VMEM((_L, _BPW), jnp.int32),
        pltpu.VMEM((_L, _BPW), jnp.float32),
        pltpu.VMEM((_BPW,), jnp.float32),
        pltpu.SemaphoreType.DMA,
    ],
    compiler_params=pltpu.CompilerParams(
        needs_layout_passes=False, use_tc_tiling_on_sc=False),
)
def _sc_pool(idx_hbm, tdot_hbm, out_hbm, idx_v, val_v, out_v, sem):
    wid = lax.axis_index("s") * _NC + lax.axis_index("c")
    base = wid * _BPW

    idx_copies = [
        pltpu.async_copy(idx_hbm.at[l, wid], idx_v.at[l], sem)
        for l in range(_L)
    ]
    for cp in idx_copies:
        cp.wait()

    copies = [
        pltpu.async_copy(tdot_hbm.at[idx_v.at[l]], val_v.at[l], sem)
        for l in range(_L)
    ]
    for cp in copies:
        cp.wait()

    for c in range(_BPW // 16):
        sl = pl.ds(c * 16, 16)
        acc = val_v[0, sl]
        for l in range(1, _L):
            acc = acc + val_v[l, sl]
        out_v[sl] = 1.0 / (1.0 + jnp.exp(-acc))

    pltpu.sync_copy(out_v, out_hbm.at[pl.ds(base, _BPW)])


def kernel(x, table, W, b):
    table_t = table.astype(jnp.float32).T
    tdot = _tdot(table_t, W.astype(jnp.float32), b.astype(jnp.float32))
    idx = x.astype(jnp.int32).T.reshape(_L, _NW, _BPW)
    out = _sc_pool(idx, tdot.reshape(-1))
    return out.reshape(_B, 1)


# speedup vs baseline: 7.5346x; 2.7087x over previous
import functools

import jax
import jax.numpy as jnp
from jax import lax
from jax.experimental import pallas as pl
from jax.experimental.pallas import tpu as pltpu
from jax.experimental.pallas import tpu_sc as plsc

_B = 4096
_L = 20
_D = 16
_NC = 2
_NS = 16
_NW = _NC * _NS
_BPW = _B // _NW

_V = 1000000
_CPB = 32768
_GRID = (_V + _CPB - 1) // _CPB
_ROWS = _GRID * (_CPB // 128)

_mesh = plsc.VectorSubcoreMesh(core_axis_name="c", subcore_axis_name="s")


def _tdot_body(tt_ref, w_ref, b_ref, out_ref):
    acc = tt_ref[...] * w_ref[...]
    s = jnp.sum(acc, axis=0, keepdims=True)
    bias = b_ref[0] * (1.0 / _L)
    for r in range(_CPB // 128):
        out_ref[r:r + 1, :] = s[:, r * 128:(r + 1) * 128] + bias


def _tdot(table_t, W, b):
    return pl.pallas_call(
        _tdot_body,
        grid=(_GRID,),
        in_specs=[
            pl.BlockSpec((_D, _CPB), lambda g: (0, g)),
            pl.BlockSpec((_D, 1), lambda g: (0, 0)),
            pl.BlockSpec(memory_space=pltpu.SMEM),
        ],
        out_specs=pl.BlockSpec((_CPB // 128, 128), lambda g: (g, 0)),
        out_shape=jax.ShapeDtypeStruct((_ROWS, 128), jnp.float32),
    )(table_t, W, b)


@functools.partial(
    pl.kernel,
    mesh=_mesh,
    out_type=jax.ShapeDtypeStruct((_B,), jnp.float32),
    scratch_types=[
        pltpu.VMEM((_L, _BPW), jnp.int32),
        pltpu.VMEM((_L, _BPW), jnp.float32),
        pltpu.VMEM((_BPW,), jnp.float32),
        pltpu.SemaphoreType.DMA,
    ],
    compiler_params=pltpu.CompilerParams(
        needs_layout_passes=False, use_tc_tiling_on_sc=False),
)
def _sc_pool(idx_hbm, tdot_hbm, out_hbm, idx_v, val_v, out_v, sem):
    wid = lax.axis_index("s") * _NC + lax.axis_index("c")
    base = wid * _BPW

    idx_copies = [
        pltpu.async_copy(idx_hbm.at[l, wid], idx_v.at[l], sem)
        for l in range(_L)
    ]
    for cp in idx_copies:
        cp.wait()

    copies = [
        pltpu.async_copy(tdot_hbm.at[idx_v.at[l]], val_v.at[l], sem)
        for l in range(_L)
    ]
    for cp in copies:
        cp.wait()

    for c in range(_BPW // 16):
        sl = pl.ds(c * 16, 16)
        acc = val_v[0, sl]
        for l in range(1, _L):
            acc = acc + val_v[l, sl]
        out_v[sl] = 1.0 / (1.0 + jnp.exp(-acc))

    pltpu.sync_copy(out_v, out_hbm.at[pl.ds(base, _BPW)])


def kernel(x, table, W, b):
    table_t = table.astype(jnp.float32).T
    tdot = _tdot(table_t, W.astype(jnp.float32), b.astype(jnp.float32))
    idx = x.astype(jnp.int32).T.reshape(_L, _NW, _BPW)
    out = _sc_pool(idx, tdot.reshape(-1))
    return out.reshape(_B, 1)


# speedup vs baseline: 8.6018x; 1.1416x over previous
import functools

import jax
import jax.numpy as jnp
from jax import lax
from jax.experimental import pallas as pl
from jax.experimental.pallas import tpu as pltpu
from jax.experimental.pallas import tpu_sc as plsc

_B = 4096
_L = 20
_D = 16
_NC = 2
_NS = 16
_NW = _NC * _NS
_BPW = _B // _NW

_V = 1000000
_CPB = 65536
_GRID = (_V + _CPB - 1) // _CPB
_ROWS = _GRID * (_CPB // 128)

_mesh = plsc.VectorSubcoreMesh(core_axis_name="c", subcore_axis_name="s")


def _tdot_body(tt_ref, w_ref, b_ref, out_ref):
    acc = tt_ref[...] * w_ref[...]
    s = jnp.sum(acc, axis=0, keepdims=True)
    bias = b_ref[0] * (1.0 / _L)
    for r in range(_CPB // 128):
        out_ref[r:r + 1, :] = s[:, r * 128:(r + 1) * 128] + bias


def _tdot(table_t, W, b):
    return pl.pallas_call(
        _tdot_body,
        grid=(_GRID,),
        in_specs=[
            pl.BlockSpec((_D, _CPB), lambda g: (0, g)),
            pl.BlockSpec((_D, 1), lambda g: (0, 0)),
            pl.BlockSpec(memory_space=pltpu.SMEM),
        ],
        out_specs=pl.BlockSpec((_CPB // 128, 128), lambda g: (g, 0)),
        out_shape=jax.ShapeDtypeStruct((_ROWS, 128), jnp.float32),
    )(table_t, W, b)


@functools.partial(
    pl.kernel,
    mesh=_mesh,
    out_type=jax.ShapeDtypeStruct((_B,), jnp.float32),
    scratch_types=[
        pltpu.VMEM((_L, _BPW), jnp.int32),
        pltpu.VMEM((_L, _BPW), jnp.float32),
        pltpu.VMEM((_BPW,), jnp.float32),
        pltpu.SemaphoreType.DMA,
    ],
    compiler_params=pltpu.CompilerParams(
        needs_layout_passes=False, use_tc_tiling_on_sc=False),
)
def _sc_pool(idx_hbm, tdot_hbm, out_hbm, idx_v, val_v, out_v, sem):
    wid = lax.axis_index("s") * _NC + lax.axis_index("c")
    base = wid * _BPW

    idx_copies = [
        pltpu.async_copy(idx_hbm.at[l, wid], idx_v.at[l], sem)
        for l in range(_L)
    ]
    for cp in idx_copies:
        cp.wait()

    copies = [
        pltpu.async_copy(tdot_hbm.at[idx_v.at[l]], val_v.at[l], sem)
        for l in range(_L)
    ]
    for cp in copies:
        cp.wait()

    for c in range(_BPW // 16):
        sl = pl.ds(c * 16, 16)
        acc = val_v[0, sl]
        for l in range(1, _L):
            acc = acc + val_v[l, sl]
        out_v[sl] = 1.0 / (1.0 + jnp.exp(-acc))

    pltpu.sync_copy(out_v, out_hbm.at[pl.ds(base, _BPW)])


def kernel(x, table, W, b):
    table_t = table.astype(jnp.float32).T
    tdot = _tdot(table_t, W.astype(jnp.float32), b.astype(jnp.float32))
    idx = x.astype(jnp.int32).T.reshape(_L, _NW, _BPW)
    out = _sc_pool(idx, tdot.reshape(-1))
    return out.reshape(_B, 1)


# speedup vs baseline: 9.2126x; 1.0710x over previous
import functools

import jax
import jax.numpy as jnp
from jax import lax
from jax.experimental import pallas as pl
from jax.experimental.pallas import tpu as pltpu
from jax.experimental.pallas import tpu_sc as plsc

_B = 4096
_L = 20
_D = 16
_NC = 2
_NS = 16
_NW = _NC * _NS
_BPW = _B // _NW

_V = 1000000
_CPB = 131072
_GRID = (_V + _CPB - 1) // _CPB
_ROWS = _GRID * (_CPB // 128)

_mesh = plsc.VectorSubcoreMesh(core_axis_name="c", subcore_axis_name="s")


def _tdot_body(tt_ref, w_ref, b_ref, out_ref):
    acc = tt_ref[...] * w_ref[...]
    s = jnp.sum(acc, axis=0, keepdims=True)
    bias = b_ref[0] * (1.0 / _L)
    for r in range(_CPB // 128):
        out_ref[r:r + 1, :] = s[:, r * 128:(r + 1) * 128] + bias


def _tdot(table_t, W, b):
    return pl.pallas_call(
        _tdot_body,
        grid=(_GRID,),
        in_specs=[
            pl.BlockSpec((_D, _CPB), lambda g: (0, g)),
            pl.BlockSpec((_D, 1), lambda g: (0, 0)),
            pl.BlockSpec(memory_space=pltpu.SMEM),
        ],
        out_specs=pl.BlockSpec((_CPB // 128, 128), lambda g: (g, 0)),
        out_shape=jax.ShapeDtypeStruct((_ROWS, 128), jnp.float32),
    )(table_t, W, b)


@functools.partial(
    pl.kernel,
    mesh=_mesh,
    out_type=jax.ShapeDtypeStruct((_B,), jnp.float32),
    scratch_types=[
        pltpu.VMEM((_L, _BPW), jnp.int32),
        pltpu.VMEM((_L, _BPW), jnp.float32),
        pltpu.VMEM((_BPW,), jnp.float32),
        pltpu.SemaphoreType.DMA,
    ],
    compiler_params=pltpu.CompilerParams(
        needs_layout_passes=False, use_tc_tiling_on_sc=False),
)
def _sc_pool(idx_hbm, tdot_hbm, out_hbm, idx_v, val_v, out_v, sem):
    wid = lax.axis_index("s") * _NC + lax.axis_index("c")
    base = wid * _BPW

    idx_copies = [
        pltpu.async_copy(idx_hbm.at[l, wid], idx_v.at[l], sem)
        for l in range(_L)
    ]
    for cp in idx_copies:
        cp.wait()

    copies = [
        pltpu.async_copy(tdot_hbm.at[idx_v.at[l]], val_v.at[l], sem)
        for l in range(_L)
    ]
    for cp in copies:
        cp.wait()

    for c in range(_BPW // 16):
        sl = pl.ds(c * 16, 16)
        acc = val_v[0, sl]
        for l in range(1, _L):
            acc = acc + val_v[l, sl]
        out_v[sl] = 1.0 / (1.0 + jnp.exp(-acc))

    pltpu.sync_copy(out_v, out_hbm.at[pl.ds(base, _BPW)])


def kernel(x, table, W, b):
    table_t = table.astype(jnp.float32).T
    tdot = _tdot(table_t, W.astype(jnp.float32), b.astype(jnp.float32))
    idx = x.astype(jnp.int32).T.reshape(_L, _NW, _BPW)
    out = _sc_pool(idx, tdot.reshape(-1))
    return out.reshape(_B, 1)


# speedup vs baseline: 9.2344x; 1.0024x over previous
import functools

import jax
import jax.numpy as jnp
from jax import lax
from jax.experimental import pallas as pl
from jax.experimental.pallas import tpu as pltpu
from jax.experimental.pallas import tpu_sc as plsc

_B = 4096
_L = 20
_D = 16
_NC = 2
_NS = 16
_NW = _NC * _NS
_BPW = _B // _NW

_V = 1000000
_CPB = 131072
_GRID = (_V + _CPB - 1) // _CPB
_ROWS = _GRID * (_CPB // 128)

_mesh = plsc.VectorSubcoreMesh(core_axis_name="c", subcore_axis_name="s")


def _tdot_body(tt_ref, w_ref, b_ref, out_ref):
    acc = tt_ref[...] * w_ref[...]
    s = jnp.sum(acc, axis=0, keepdims=True)
    bias = b_ref[0] * (1.0 / _L)
    for r in range(_CPB // 128):
        out_ref[r:r + 1, :] = s[:, r * 128:(r + 1) * 128] + bias


def _tdot(table_t, W, b):
    return pl.pallas_call(
        _tdot_body,
        grid=(_GRID,),
        in_specs=[
            pl.BlockSpec((_D, _CPB), lambda g: (0, g)),
            pl.BlockSpec((_D, 1), lambda g: (0, 0)),
            pl.BlockSpec(memory_space=pltpu.SMEM),
        ],
        out_specs=pl.BlockSpec((_CPB // 128, 128), lambda g: (g, 0)),
        out_shape=jax.ShapeDtypeStruct((_ROWS, 128), jnp.float32),
    )(table_t, W, b)


@functools.partial(
    pl.kernel,
    mesh=_mesh,
    out_type=jax.ShapeDtypeStruct((_B,), jnp.float32),
    scratch_types=[
        pltpu.VMEM((_L, _BPW), jnp.int32),
        pltpu.VMEM((_L, _BPW), jnp.float32),
        pltpu.VMEM((_BPW,), jnp.float32),
        pltpu.SemaphoreType.DMA,
        pltpu.SemaphoreType.DMA,
    ],
    compiler_params=pltpu.CompilerParams(
        needs_layout_passes=False, use_tc_tiling_on_sc=False),
)
def _sc_pool(idx_hbm, tdot_hbm, out_hbm, idx_v, val_v, out_v, sem, idx_sem):
    wid = lax.axis_index("s") * _NC + lax.axis_index("c")
    base = wid * _BPW

    idx_copies = [
        pltpu.async_copy(idx_hbm.at[l, wid], idx_v.at[l], idx_sem)
        for l in range(_L)
    ]
    copies = []
    for l in range(_L):
        idx_copies[l].wait()
        copies.append(
            pltpu.async_copy(tdot_hbm.at[idx_v.at[l]], val_v.at[l], sem))
    for cp in copies:
        cp.wait()

    for c in range(_BPW // 16):
        sl = pl.ds(c * 16, 16)
        acc = val_v[0, sl]
        for l in range(1, _L):
            acc = acc + val_v[l, sl]
        out_v[sl] = 1.0 / (1.0 + jnp.exp(-acc))

    pltpu.sync_copy(out_v, out_hbm.at[pl.ds(base, _BPW)])


def kernel(x, table, W, b):
    table_t = table.astype(jnp.float32).T
    tdot = _tdot(table_t, W.astype(jnp.float32), b.astype(jnp.float32))
    idx = x.astype(jnp.int32).T.reshape(_L, _NW, _BPW)
    out = _sc_pool(idx, tdot.reshape(-1))
    return out.reshape(_B, 1)
